# baseline (speedup 1.0000x reference)
import functools

import jax
import jax.numpy as jnp
from jax.experimental import pallas as pl
from jax.experimental.pallas import tpu as pltpu


def _router_moe_body(x_ref, wg_ref, bg_ref, w1_ref, b1_ref, w2_ref, b2_ref,
                     out_ref, w_scr):
    e = pl.program_id(1)
    i2 = pl.program_id(2)

    @pl.when(jnp.logical_and(e == 0, i2 == 0))
    def _():
        x = x_ref[...]
        logits = jnp.dot(x, wg_ref[...], preferred_element_type=jnp.float32)
        logits = logits + bg_ref[...]
        m = jnp.max(logits, axis=-1, keepdims=True)
        p = jnp.exp(logits - m)
        p = p / jnp.sum(p, axis=-1, keepdims=True)
        top1 = jnp.max(p, axis=-1, keepdims=True)
        is1 = p == top1
        p2 = jnp.where(is1, -jnp.inf, p)
        top2 = jnp.max(p2, axis=-1, keepdims=True)
        is2 = p2 == top2
        denom = top1 + top2 + 1e-9
        w_scr[...] = jnp.where(is1 | is2, p, 0.0) / denom
        out_ref[...] = jnp.zeros_like(out_ref)

    x = x_ref[...].astype(jnp.bfloat16)
    h = jax.nn.gelu(
        jnp.dot(x, w1_ref[0].astype(jnp.bfloat16),
                preferred_element_type=jnp.float32) + b1_ref[0])
    y = jnp.dot(h.astype(jnp.bfloat16), w2_ref[0].astype(jnp.bfloat16),
                preferred_element_type=jnp.float32)
    y = y + jnp.where(i2 == 0, 1.0, 0.0) * b2_ref[0]
    lane = jax.lax.broadcasted_iota(jnp.int32, w_scr.shape, 1)
    w_col = jnp.sum(jnp.where(lane == e, w_scr[...], 0.0), axis=1, keepdims=True)
    out_ref[...] += w_col * y


def _shared_body(x_ref, w1_ref, b1_ref, w2_ref, b2_ref, moe_ref, out_ref):
    i2 = pl.program_id(1)

    @pl.when(i2 == 0)
    def _():
        out_ref[...] = x_ref[...] + moe_ref[...] + b2_ref[...]

    h = jax.nn.gelu(
        jnp.dot(x_ref[...].astype(jnp.bfloat16),
                w1_ref[...].astype(jnp.bfloat16),
                preferred_element_type=jnp.float32)
        + b1_ref[...])
    out_ref[...] += jnp.dot(h.astype(jnp.bfloat16),
                            w2_ref[...].astype(jnp.bfloat16),
                            preferred_element_type=jnp.float32)


def _moe_dense(xs, Wg, bg, Ws1, bs1, Ws2, bs2, We1, be1, We2, be2,
               bt, ib):
    S, D = xs.shape
    E = Wg.shape[1]
    I = Ws1.shape[1]
    T = S // bt
    NI = I // ib
    moe = pl.pallas_call(
        _router_moe_body,
        grid=(T, E, NI),
        in_specs=[
            pl.BlockSpec((bt, D), lambda t, e, i: (t, 0)),
            pl.BlockSpec((D, E), lambda t, e, i: (0, 0)),
            pl.BlockSpec((1, E), lambda t, e, i: (0, 0)),
            pl.BlockSpec((1, D, ib), lambda t, e, i: (e, 0, i)),
            pl.BlockSpec((1, 1, ib), lambda t, e, i: (e, 0, i)),
            pl.BlockSpec((1, ib, D), lambda t, e, i: (e, i, 0)),
            pl.BlockSpec((1, 1, D), lambda t, e, i: (e, 0, 0)),
        ],
        out_specs=pl.BlockSpec((bt, D), lambda t, e, i: (t, 0)),
        out_shape=jax.ShapeDtypeStruct((S, D), jnp.float32),
        scratch_shapes=[pltpu.VMEM((bt, E), jnp.float32)],
    )(xs, Wg, bg.reshape(1, E), We1, be1.reshape(E, 1, I), We2,
      be2.reshape(E, 1, D))
    out = pl.pallas_call(
        _shared_body,
        grid=(T, NI),
        in_specs=[
            pl.BlockSpec((bt, D), lambda t, i: (t, 0)),
            pl.BlockSpec((D, ib), lambda t, i: (0, i)),
            pl.BlockSpec((1, ib), lambda t, i: (0, i)),
            pl.BlockSpec((ib, D), lambda t, i: (i, 0)),
            pl.BlockSpec((1, D), lambda t, i: (0, 0)),
            pl.BlockSpec((bt, D), lambda t, i: (t, 0)),
        ],
        out_specs=pl.BlockSpec((bt, D), lambda t, i: (t, 0)),
        out_shape=jax.ShapeDtypeStruct((S, D), jnp.float32),
    )(xs, Ws1, bs1.reshape(1, I), Ws2, bs2.reshape(1, D), moe)
    return out


def kernel(x, Wg, bg, Ws1, bs1, Ws2, bs2, We1, be1, We2, be2):
    B, S, D = x.shape
    xs = x.reshape(S, D)
    bt = min(S, 1024)
    ib = min(Ws1.shape[1], 512)
    out = _moe_dense(xs, Wg, bg, Ws1, bs1, Ws2, bs2, We1, be1, We2, be2, bt, ib)
    return out.reshape(B, S, D)


# speedup vs baseline: 1.1830x; 1.1830x over previous
import functools

import jax
import jax.numpy as jnp
from jax import lax
from jax.experimental import pallas as pl
from jax.experimental.pallas import tpu as pltpu
from jax.experimental.pallas import tpu_sc as plsc

M = 512
NB = 15
NBM = NB * M
BLKW = 16


def _router_shared_body(x_ref, wg_ref, bg_ref, w1_ref, b1_ref, w2_ref, b2_ref,
                        out_ref, d0_ref, d1_ref, gw0_ref, gw1_ref,
                        blke_ref, blka_ref):
    i2 = pl.program_id(0)

    @pl.when(i2 == 0)
    def _():
        x = x_ref[...]
        S = x.shape[0]
        E = wg_ref.shape[1]
        logits = jnp.dot(x, wg_ref[...], preferred_element_type=jnp.float32)
        logits = logits + bg_ref[...]
        m = jnp.max(logits, axis=-1, keepdims=True)
        p = jnp.exp(logits - m)
        p = p / jnp.sum(p, axis=-1, keepdims=True)
        top1 = jnp.max(p, axis=-1, keepdims=True)
        is1 = p == top1
        p2 = jnp.where(is1, -jnp.inf, p)
        top2 = jnp.max(p2, axis=-1, keepdims=True)
        is2 = p2 == top2
        denom = top1 + top2 + 1e-9
        gw0_ref[...] = top1 / denom
        gw1_ref[...] = top2 / denom

        m0 = is1.astype(jnp.float32)
        m1 = is2.astype(jnp.float32)
        r = lax.broadcasted_iota(jnp.int32, (128, 128), 0)
        c = lax.broadcasted_iota(jnp.int32, (128, 128), 1)
        lstrict = (r < c).astype(jnp.float32)

        def ranks(mm):
            run = jnp.zeros((1, E), jnp.float32)
            parts = []
            for i in range(S // 128):
                ch = mm[128 * i:128 * (i + 1), :]
                pre = jnp.dot(jnp.transpose(lstrict),
                              ch, preferred_element_type=jnp.float32)
                parts.append(pre + run)
                run = run + jnp.sum(ch, axis=0, keepdims=True)
            return jnp.concatenate(parts, axis=0), run

        rank0, cnt0 = ranks(m0)
        rank1, cnt1 = ranks(m1)
        rank1 = rank1 + cnt0
        counts = cnt0 + cnt1
        nblk = jnp.ceil(counts / M)
        l8r = lax.broadcasted_iota(jnp.int32, (E, E), 0)
        l8c = lax.broadcasted_iota(jnp.int32, (E, E), 1)
        l8 = (l8r < l8c).astype(jnp.float32)
        excl_blk = jnp.dot(nblk, l8, preferred_element_type=jnp.float32)
        offs = M * excl_blk
        d0 = offs + rank0
        d1 = offs + rank1
        d0_ref[...] = jnp.sum(jnp.where(is1, d0, 0.0), axis=-1,
                              keepdims=True).astype(jnp.int32)
        d1_ref[...] = jnp.sum(jnp.where(is2, d1, 0.0), axis=-1,
                              keepdims=True).astype(jnp.int32)

        end_blk = excl_blk + nblk
        total_blk = end_blk[:, E - 1:E]
        biota = lax.broadcasted_iota(
            jnp.int32, (1, blke_ref.shape[1]), 1).astype(jnp.float32)
        blke = jnp.zeros_like(biota)
        for e in range(E):
            end_e = end_blk[:, e:e + 1]
            blke = blke + (biota >= end_e).astype(jnp.float32)
        blke_ref[...] = jnp.minimum(blke, float(E - 1)).astype(jnp.int32)
        blka_ref[...] = (biota < total_blk).astype(jnp.int32)

        out_ref[...] = x + b2_ref[...]

    h = jax.nn.gelu(
        jnp.dot(x_ref[...], w1_ref[...], preferred_element_type=jnp.float32)
        + b1_ref[...])
    out_ref[...] += jnp.dot(h, w2_ref[...], preferred_element_type=jnp.float32)


def _expert_body(blke_ref, blka_ref, x_ref, w1_ref, b1_ref, w2_ref, b2_ref,
                 out_ref):
    b = pl.program_id(0)
    i2 = pl.program_id(1)

    @pl.when(blka_ref[b] != 0)
    def _():
        h = jax.nn.gelu(
            jnp.dot(x_ref[...], w1_ref[0], preferred_element_type=jnp.float32)
            + b1_ref[0])
        y = jnp.dot(h, w2_ref[0], preferred_element_type=jnp.float32)

        @pl.when(i2 == 0)
        def _():
            out_ref[...] = y + b2_ref[0]

        @pl.when(i2 != 0)
        def _():
            out_ref[...] += y


def _final_body(p_ref, y0_ref, y1_ref, g0_ref, g1_ref, out_ref):
    out_ref[...] = (p_ref[...] + g0_ref[...] * y0_ref[...]
                    + g1_ref[...] * y1_ref[...])


def _sc_dispatch(x, d0, d1, S, D):
    info = plsc.get_sparse_core_info()
    nw = info.num_cores * info.num_subcores
    ch = S // nw
    mesh = plsc.VectorSubcoreMesh(core_axis_name="c", subcore_axis_name="s")

    @functools.partial(
        pl.kernel, mesh=mesh,
        out_type=jax.ShapeDtypeStruct((NBM, D), jnp.float32),
        scratch_types=[
            pltpu.VMEM((ch,), jnp.int32),
            pltpu.VMEM((ch, D), jnp.float32),
            pltpu.SemaphoreType.DMA,
        ],
    )
    def disp(x_hbm, d0_hbm, d1_hbm, xs_hbm, idx_v, rows_v, sem):
        wid = lax.axis_index("s") * info.num_cores + lax.axis_index("c")
        base = wid * ch
        pltpu.sync_copy(x_hbm.at[pl.ds(base, ch)], rows_v)
        pltpu.sync_copy(d0_hbm.at[pl.ds(base, ch)], idx_v)
        pltpu.async_copy(rows_v, xs_hbm.at[idx_v], sem).wait()
        pltpu.sync_copy(d1_hbm.at[pl.ds(base, ch)], idx_v)
        pltpu.async_copy(rows_v, xs_hbm.at[idx_v], sem).wait()

    return disp(x, d0, d1)


def _sc_combine(ys, d0, d1, S, D):
    info = plsc.get_sparse_core_info()
    nw = info.num_cores * info.num_subcores
    ch = S // nw
    mesh = plsc.VectorSubcoreMesh(core_axis_name="c", subcore_axis_name="s")

    @functools.partial(
        pl.kernel, mesh=mesh,
        out_type=[jax.ShapeDtypeStruct((S, D), jnp.float32),
                  jax.ShapeDtypeStruct((S, D), jnp.float32)],
        scratch_types=[
            pltpu.VMEM((ch,), jnp.int32),
            pltpu.VMEM((ch, D), jnp.float32),
            pltpu.SemaphoreType.DMA,
        ],
    )
    def comb(ys_hbm, d0_hbm, d1_hbm, y0_hbm, y1_hbm, idx_v, rows_v, sem):
        wid = lax.axis_index("s") * info.num_cores + lax.axis_index("c")
        base = wid * ch
        pltpu.sync_copy(d0_hbm.at[pl.ds(base, ch)], idx_v)
        pltpu.async_copy(ys_hbm.at[idx_v], rows_v, sem).wait()
        pltpu.sync_copy(rows_v, y0_hbm.at[pl.ds(base, ch)])
        pltpu.sync_copy(d1_hbm.at[pl.ds(base, ch)], idx_v)
        pltpu.async_copy(ys_hbm.at[idx_v], rows_v, sem).wait()
        pltpu.sync_copy(rows_v, y1_hbm.at[pl.ds(base, ch)])

    return comb(ys, d0, d1)


def _router_shared(xs, Wg, bg, Ws1, bs1, Ws2, bs2, ib):
    S, D = xs.shape
    E = Wg.shape[1]
    I = Ws1.shape[1]
    NI = I // ib
    return pl.pallas_call(
        _router_shared_body,
        grid=(NI,),
        in_specs=[
            pl.BlockSpec((S, D), lambda i: (0, 0)),
            pl.BlockSpec((D, E), lambda i: (0, 0)),
            pl.BlockSpec((1, E), lambda i: (0, 0)),
            pl.BlockSpec((D, ib), lambda i: (0, i)),
            pl.BlockSpec((1, ib), lambda i: (0, i)),
            pl.BlockSpec((ib, D), lambda i: (i, 0)),
            pl.BlockSpec((1, D), lambda i: (0, 0)),
        ],
        out_specs=[
            pl.BlockSpec((S, D), lambda i: (0, 0)),
            pl.BlockSpec((S, 1), lambda i: (0, 0)),
            pl.BlockSpec((S, 1), lambda i: (0, 0)),
            pl.BlockSpec((S, 1), lambda i: (0, 0)),
            pl.BlockSpec((S, 1), lambda i: (0, 0)),
            pl.BlockSpec((1, BLKW), lambda i: (0, 0)),
            pl.BlockSpec((1, BLKW), lambda i: (0, 0)),
        ],
        out_shape=[
            jax.ShapeDtypeStruct((S, D), jnp.float32),
            jax.ShapeDtypeStruct((S, 1), jnp.int32),
            jax.ShapeDtypeStruct((S, 1), jnp.int32),
            jax.ShapeDtypeStruct((S, 1), jnp.float32),
            jax.ShapeDtypeStruct((S, 1), jnp.float32),
            jax.ShapeDtypeStruct((1, BLKW), jnp.int32),
            jax.ShapeDtypeStruct((1, BLKW), jnp.int32),
        ],
    )(xs, Wg, bg.reshape(1, E), Ws1, bs1.reshape(1, I), Ws2,
      bs2.reshape(1, D))


def _expert_blocks(xsorted, We1, be1, We2, be2, blke, blka, ib):
    E, D, I = We1.shape
    NI = I // ib
    grid_spec = pltpu.PrefetchScalarGridSpec(
        num_scalar_prefetch=2,
        grid=(NB, NI),
        in_specs=[
            pl.BlockSpec((M, D), lambda b, i, be, ba: (b, 0)),
            pl.BlockSpec((1, D, ib), lambda b, i, be, ba: (be[b], 0, i)),
            pl.BlockSpec((1, 1, ib), lambda b, i, be, ba: (be[b], 0, i)),
            pl.BlockSpec((1, ib, D), lambda b, i, be, ba: (be[b], i, 0)),
            pl.BlockSpec((1, 1, D), lambda b, i, be, ba: (be[b], 0, 0)),
        ],
        out_specs=pl.BlockSpec((M, D), lambda b, i, be, ba: (b, 0)),
    )
    return pl.pallas_call(
        _expert_body,
        grid_spec=grid_spec,
        out_shape=jax.ShapeDtypeStruct((NBM, D), jnp.float32),
    )(blke, blka, xsorted, We1, be1.reshape(E, 1, I), We2,
      be2.reshape(E, 1, D))


def _final(partial, y0, y1, gw0, gw1, bt):
    S, D = partial.shape
    return pl.pallas_call(
        _final_body,
        grid=(S // bt,),
        in_specs=[
            pl.BlockSpec((bt, D), lambda t: (t, 0)),
            pl.BlockSpec((bt, D), lambda t: (t, 0)),
            pl.BlockSpec((bt, D), lambda t: (t, 0)),
            pl.BlockSpec((bt, 1), lambda t: (t, 0)),
            pl.BlockSpec((bt, 1), lambda t: (t, 0)),
        ],
        out_specs=pl.BlockSpec((bt, D), lambda t: (t, 0)),
        out_shape=jax.ShapeDtypeStruct((S, D), jnp.float32),
    )(partial, y0, y1, gw0, gw1)


def kernel(x, Wg, bg, Ws1, bs1, Ws2, bs2, We1, be1, We2, be2):
    B, S, D = x.shape
    I = Ws1.shape[1]
    xs = x.reshape(S, D)
    partial, d0, d1, gw0, gw1, blke, blka = _router_shared(
        xs, Wg, bg, Ws1, bs1, Ws2, bs2, min(I, 512))
    xsorted = _sc_dispatch(xs, d0.reshape(S), d1.reshape(S), S, D)
    ysorted = _expert_blocks(xsorted, We1, be1, We2, be2,
                             blke.reshape(BLKW), blka.reshape(BLKW),
                             min(I, 1024))
    y0, y1 = _sc_combine(ysorted, d0.reshape(S), d1.reshape(S), S, D)
    out = _final(partial, y0, y1, gw0, gw1, min(S, 1024))
    return out.reshape(B, S, D)


# speedup vs baseline: 1.3192x; 1.1151x over previous
import functools

import jax
import jax.numpy as jnp
from jax import lax
from jax.experimental import pallas as pl
from jax.experimental.pallas import tpu as pltpu
from jax.experimental.pallas import tpu_sc as plsc

M = 512
NB = 15
NBM = NB * M
BLKW = 16


def _router_shared_body(x_ref, wg_ref, bg_ref, w1_ref, b1_ref, w2_ref, b2_ref,
                        out_ref, d0_ref, d1_ref, gw0_ref, gw1_ref,
                        blke_ref, blka_ref):
    i2 = pl.program_id(0)

    @pl.when(i2 == 0)
    def _():
        x = x_ref[...]
        S = x.shape[0]
        E = wg_ref.shape[1]
        logits = jnp.dot(x, wg_ref[...], preferred_element_type=jnp.float32)
        logits = logits + bg_ref[...]
        m = jnp.max(logits, axis=-1, keepdims=True)
        p = jnp.exp(logits - m)
        p = p / jnp.sum(p, axis=-1, keepdims=True)
        top1 = jnp.max(p, axis=-1, keepdims=True)
        is1 = p == top1
        p2 = jnp.where(is1, -jnp.inf, p)
        top2 = jnp.max(p2, axis=-1, keepdims=True)
        is2 = p2 == top2
        denom = top1 + top2 + 1e-9
        gw0_ref[...] = top1 / denom
        gw1_ref[...] = top2 / denom

        m0 = is1.astype(jnp.float32)
        m1 = is2.astype(jnp.float32)
        r = lax.broadcasted_iota(jnp.int32, (128, 128), 0)
        c = lax.broadcasted_iota(jnp.int32, (128, 128), 1)
        lstrict = (r < c).astype(jnp.float32)

        def ranks(mm):
            run = jnp.zeros((1, E), jnp.float32)
            parts = []
            for i in range(S // 128):
                ch = mm[128 * i:128 * (i + 1), :]
                pre = jnp.dot(jnp.transpose(lstrict),
                              ch, preferred_element_type=jnp.float32)
                parts.append(pre + run)
                run = run + jnp.sum(ch, axis=0, keepdims=True)
            return jnp.concatenate(parts, axis=0), run

        rank0, cnt0 = ranks(m0)
        rank1, cnt1 = ranks(m1)
        rank1 = rank1 + cnt0
        counts = cnt0 + cnt1
        nblk = jnp.ceil(counts / M)
        l8r = lax.broadcasted_iota(jnp.int32, (E, E), 0)
        l8c = lax.broadcasted_iota(jnp.int32, (E, E), 1)
        l8 = (l8r < l8c).astype(jnp.float32)
        excl_blk = jnp.dot(nblk, l8, preferred_element_type=jnp.float32)
        offs = M * excl_blk
        d0 = offs + rank0
        d1 = offs + rank1
        d0_ref[...] = jnp.sum(jnp.where(is1, d0, 0.0), axis=-1,
                              keepdims=True).astype(jnp.int32)
        d1_ref[...] = jnp.sum(jnp.where(is2, d1, 0.0), axis=-1,
                              keepdims=True).astype(jnp.int32)

        end_blk = excl_blk + nblk
        total_blk = end_blk[:, E - 1:E]
        biota = lax.broadcasted_iota(
            jnp.int32, (1, blke_ref.shape[1]), 1).astype(jnp.float32)
        blke = jnp.zeros_like(biota)
        for e in range(E):
            end_e = end_blk[:, e:e + 1]
            blke = blke + (biota >= end_e).astype(jnp.float32)
        blke_ref[...] = jnp.minimum(blke, float(E - 1)).astype(jnp.int32)
        blka_ref[...] = (biota < total_blk).astype(jnp.int32)

        out_ref[...] = x + b2_ref[...]

    h = jax.nn.gelu(
        jnp.dot(x_ref[...].astype(jnp.bfloat16),
                w1_ref[...].astype(jnp.bfloat16),
                preferred_element_type=jnp.float32)
        + b1_ref[...])
    out_ref[...] += jnp.dot(h.astype(jnp.bfloat16),
                            w2_ref[...].astype(jnp.bfloat16),
                            preferred_element_type=jnp.float32)


def _expert_body(blke_ref, blka_ref, x_ref, w1_ref, b1_ref, w2_ref, b2_ref,
                 out_ref):
    b = pl.program_id(0)

    @pl.when(blka_ref[b] != 0)
    def _():
        h = jax.nn.gelu(
            jnp.dot(x_ref[...].astype(jnp.bfloat16),
                    w1_ref[0].astype(jnp.bfloat16),
                    preferred_element_type=jnp.float32)
            + b1_ref[0])
        y = jnp.dot(h.astype(jnp.bfloat16), w2_ref[0].astype(jnp.bfloat16),
                    preferred_element_type=jnp.float32)
        out_ref[...] = y + b2_ref[0]


def _final_body(p_ref, y0_ref, y1_ref, g0_ref, g1_ref, out_ref):
    out_ref[...] = (p_ref[...] + g0_ref[...] * y0_ref[...]
                    + g1_ref[...] * y1_ref[...])


def _sc_dispatch(x, d0, d1, S, D):
    info = plsc.get_sparse_core_info()
    nw = info.num_cores * info.num_subcores
    ch = S // nw
    mesh = plsc.VectorSubcoreMesh(core_axis_name="c", subcore_axis_name="s")

    @functools.partial(
        pl.kernel, mesh=mesh,
        out_type=jax.ShapeDtypeStruct((NBM, D), jnp.float32),
        scratch_types=[
            pltpu.VMEM((ch,), jnp.int32),
            pltpu.VMEM((ch, D), jnp.float32),
            pltpu.SemaphoreType.DMA,
        ],
    )
    def disp(x_hbm, d0_hbm, d1_hbm, xs_hbm, idx_v, rows_v, sem):
        wid = lax.axis_index("s") * info.num_cores + lax.axis_index("c")
        base = wid * ch
        pltpu.sync_copy(x_hbm.at[pl.ds(base, ch)], rows_v)
        pltpu.sync_copy(d0_hbm.at[pl.ds(base, ch)], idx_v)
        pltpu.async_copy(rows_v, xs_hbm.at[idx_v], sem).wait()
        pltpu.sync_copy(d1_hbm.at[pl.ds(base, ch)], idx_v)
        pltpu.async_copy(rows_v, xs_hbm.at[idx_v], sem).wait()

    return disp(x, d0, d1)


def _sc_combine(ys, d0, d1, S, D):
    info = plsc.get_sparse_core_info()
    nw = info.num_cores * info.num_subcores
    ch = S // nw
    mesh = plsc.VectorSubcoreMesh(core_axis_name="c", subcore_axis_name="s")

    @functools.partial(
        pl.kernel, mesh=mesh,
        out_type=[jax.ShapeDtypeStruct((S, D), jnp.float32),
                  jax.ShapeDtypeStruct((S, D), jnp.float32)],
        scratch_types=[
            pltpu.VMEM((ch,), jnp.int32),
            pltpu.VMEM((ch, D), jnp.float32),
            pltpu.SemaphoreType.DMA,
        ],
    )
    def comb(ys_hbm, d0_hbm, d1_hbm, y0_hbm, y1_hbm, idx_v, rows_v, sem):
        wid = lax.axis_index("s") * info.num_cores + lax.axis_index("c")
        base = wid * ch
        pltpu.sync_copy(d0_hbm.at[pl.ds(base, ch)], idx_v)
        pltpu.async_copy(ys_hbm.at[idx_v], rows_v, sem).wait()
        pltpu.sync_copy(rows_v, y0_hbm.at[pl.ds(base, ch)])
        pltpu.sync_copy(d1_hbm.at[pl.ds(base, ch)], idx_v)
        pltpu.async_copy(ys_hbm.at[idx_v], rows_v, sem).wait()
        pltpu.sync_copy(rows_v, y1_hbm.at[pl.ds(base, ch)])

    return comb(ys, d0, d1)


def _router_shared(xs, Wg, bg, Ws1, bs1, Ws2, bs2, ib):
    S, D = xs.shape
    E = Wg.shape[1]
    I = Ws1.shape[1]
    NI = I // ib
    return pl.pallas_call(
        _router_shared_body,
        grid=(NI,),
        in_specs=[
            pl.BlockSpec((S, D), lambda i: (0, 0)),
            pl.BlockSpec((D, E), lambda i: (0, 0)),
            pl.BlockSpec((1, E), lambda i: (0, 0)),
            pl.BlockSpec((D, ib), lambda i: (0, i)),
            pl.BlockSpec((1, ib), lambda i: (0, i)),
            pl.BlockSpec((ib, D), lambda i: (i, 0)),
            pl.BlockSpec((1, D), lambda i: (0, 0)),
        ],
        out_specs=[
            pl.BlockSpec((S, D), lambda i: (0, 0)),
            pl.BlockSpec((S, 1), lambda i: (0, 0)),
            pl.BlockSpec((S, 1), lambda i: (0, 0)),
            pl.BlockSpec((S, 1), lambda i: (0, 0)),
            pl.BlockSpec((S, 1), lambda i: (0, 0)),
            pl.BlockSpec((1, BLKW), lambda i: (0, 0)),
            pl.BlockSpec((1, BLKW), lambda i: (0, 0)),
        ],
        out_shape=[
            jax.ShapeDtypeStruct((S, D), jnp.float32),
            jax.ShapeDtypeStruct((S, 1), jnp.int32),
            jax.ShapeDtypeStruct((S, 1), jnp.int32),
            jax.ShapeDtypeStruct((S, 1), jnp.float32),
            jax.ShapeDtypeStruct((S, 1), jnp.float32),
            jax.ShapeDtypeStruct((1, BLKW), jnp.int32),
            jax.ShapeDtypeStruct((1, BLKW), jnp.int32),
        ],
    )(xs, Wg, bg.reshape(1, E), Ws1, bs1.reshape(1, I), Ws2,
      bs2.reshape(1, D))


def _expert_blocks(xsorted, We1, be1, We2, be2, blke, blka):
    E, D, I = We1.shape
    grid_spec = pltpu.PrefetchScalarGridSpec(
        num_scalar_prefetch=2,
        grid=(NB,),
        in_specs=[
            pl.BlockSpec((M, D), lambda b, be, ba: (b, 0)),
            pl.BlockSpec((1, D, I), lambda b, be, ba: (be[b], 0, 0)),
            pl.BlockSpec((1, 1, I), lambda b, be, ba: (be[b], 0, 0)),
            pl.BlockSpec((1, I, D), lambda b, be, ba: (be[b], 0, 0)),
            pl.BlockSpec((1, 1, D), lambda b, be, ba: (be[b], 0, 0)),
        ],
        out_specs=pl.BlockSpec((M, D), lambda b, be, ba: (b, 0)),
    )
    return pl.pallas_call(
        _expert_body,
        grid_spec=grid_spec,
        out_shape=jax.ShapeDtypeStruct((NBM, D), jnp.float32),
    )(blke, blka, xsorted, We1, be1.reshape(E, 1, I), We2,
      be2.reshape(E, 1, D))


def _final(partial, y0, y1, gw0, gw1, bt):
    S, D = partial.shape
    return pl.pallas_call(
        _final_body,
        grid=(S // bt,),
        in_specs=[
            pl.BlockSpec((bt, D), lambda t: (t, 0)),
            pl.BlockSpec((bt, D), lambda t: (t, 0)),
            pl.BlockSpec((bt, D), lambda t: (t, 0)),
            pl.BlockSpec((bt, 1), lambda t: (t, 0)),
            pl.BlockSpec((bt, 1), lambda t: (t, 0)),
        ],
        out_specs=pl.BlockSpec((bt, D), lambda t: (t, 0)),
        out_shape=jax.ShapeDtypeStruct((S, D), jnp.float32),
    )(partial, y0, y1, gw0, gw1)


def kernel(x, Wg, bg, Ws1, bs1, Ws2, bs2, We1, be1, We2, be2):
    B, S, D = x.shape
    I = Ws1.shape[1]
    xs = x.reshape(S, D)
    partial, d0, d1, gw0, gw1, blke, blka = _router_shared(
        xs, Wg, bg, Ws1, bs1, Ws2, bs2, min(I, 512))
    xsorted = _sc_dispatch(xs, d0.reshape(S), d1.reshape(S), S, D)
    ysorted = _expert_blocks(xsorted, We1, be1, We2, be2,
                             blke.reshape(BLKW), blka.reshape(BLKW))
    y0, y1 = _sc_combine(ysorted, d0.reshape(S), d1.reshape(S), S, D)
    out = _final(partial, y0, y1, gw0, gw1, min(S, 1024))
    return out.reshape(B, S, D)


# speedup vs baseline: 1.3829x; 1.0483x over previous
import functools

import jax
import jax.numpy as jnp
from jax import lax
from jax.experimental import pallas as pl
from jax.experimental.pallas import tpu as pltpu
from jax.experimental.pallas import tpu_sc as plsc

M = 512
NB = 15
NBM = NB * M
BLKW = 16


def _router_body(x_ref, wg_ref, bg_ref,
                 d0_ref, d1_ref, gw0_ref, gw1_ref, blke_ref, blka_ref):
    x = x_ref[...]
    S = x.shape[0]
    E = wg_ref.shape[1]
    logits = jnp.dot(x, wg_ref[...], preferred_element_type=jnp.float32)
    logits = logits + bg_ref[...]
    m = jnp.max(logits, axis=-1, keepdims=True)
    p = jnp.exp(logits - m)
    p = p / jnp.sum(p, axis=-1, keepdims=True)
    top1 = jnp.max(p, axis=-1, keepdims=True)
    is1 = p == top1
    p2 = jnp.where(is1, -jnp.inf, p)
    top2 = jnp.max(p2, axis=-1, keepdims=True)
    is2 = p2 == top2
    denom = top1 + top2 + 1e-9
    gw0_ref[...] = top1 / denom
    gw1_ref[...] = top2 / denom

    m0 = is1.astype(jnp.float32)
    m1 = is2.astype(jnp.float32)
    r = lax.broadcasted_iota(jnp.int32, (128, 128), 0)
    c = lax.broadcasted_iota(jnp.int32, (128, 128), 1)
    lstrict = (r < c).astype(jnp.float32)

    def ranks(mm):
        run = jnp.zeros((1, E), jnp.float32)
        parts = []
        for i in range(S // 128):
            ch = mm[128 * i:128 * (i + 1), :]
            pre = jnp.dot(jnp.transpose(lstrict),
                          ch, preferred_element_type=jnp.float32)
            parts.append(pre + run)
            run = run + jnp.sum(ch, axis=0, keepdims=True)
        return jnp.concatenate(parts, axis=0), run

    rank0, cnt0 = ranks(m0)
    rank1, cnt1 = ranks(m1)
    rank1 = rank1 + cnt0
    counts = cnt0 + cnt1
    nblk = jnp.ceil(counts / M)
    l8r = lax.broadcasted_iota(jnp.int32, (E, E), 0)
    l8c = lax.broadcasted_iota(jnp.int32, (E, E), 1)
    l8 = (l8r < l8c).astype(jnp.float32)
    excl_blk = jnp.dot(nblk, l8, preferred_element_type=jnp.float32)
    offs = M * excl_blk
    d0 = offs + rank0
    d1 = offs + rank1
    d0_ref[...] = jnp.sum(jnp.where(is1, d0, 0.0), axis=-1,
                          keepdims=True).astype(jnp.int32)
    d1_ref[...] = jnp.sum(jnp.where(is2, d1, 0.0), axis=-1,
                          keepdims=True).astype(jnp.int32)

    end_blk = excl_blk + nblk
    total_blk = end_blk[:, E - 1:E]
    biota = lax.broadcasted_iota(
        jnp.int32, (1, blke_ref.shape[1]), 1).astype(jnp.float32)
    blke = jnp.zeros_like(biota)
    for e in range(E):
        end_e = end_blk[:, e:e + 1]
        blke = blke + (biota >= end_e).astype(jnp.float32)
    blke_ref[...] = jnp.minimum(blke, float(E - 1)).astype(jnp.int32)
    blka_ref[...] = (biota < total_blk).astype(jnp.int32)


def _shared_body(x_ref, w1_ref, b1_ref, w2_ref, b2_ref, out_ref):
    i2 = pl.program_id(0)

    @pl.when(i2 == 0)
    def _():
        out_ref[...] = x_ref[...] + b2_ref[...]

    h = jax.nn.gelu(
        jnp.dot(x_ref[...].astype(jnp.bfloat16),
                w1_ref[...].astype(jnp.bfloat16),
                preferred_element_type=jnp.float32)
        + b1_ref[...])
    out_ref[...] += jnp.dot(h.astype(jnp.bfloat16),
                            w2_ref[...].astype(jnp.bfloat16),
                            preferred_element_type=jnp.float32)


def _expert_body(blke_ref, blka_ref, x_ref, w1_ref, b1_ref, w2_ref, b2_ref,
                 out_ref):
    b = pl.program_id(0)

    @pl.when(blka_ref[b] != 0)
    def _():
        h = jax.nn.gelu(
            jnp.dot(x_ref[...].astype(jnp.bfloat16),
                    w1_ref[0].astype(jnp.bfloat16),
                    preferred_element_type=jnp.float32)
            + b1_ref[0])
        y = jnp.dot(h.astype(jnp.bfloat16), w2_ref[0].astype(jnp.bfloat16),
                    preferred_element_type=jnp.float32)
        out_ref[...] = y + b2_ref[0]


def _final_body(p_ref, y0_ref, y1_ref, g0_ref, g1_ref, out_ref):
    out_ref[...] = (p_ref[...] + g0_ref[...] * y0_ref[...]
                    + g1_ref[...] * y1_ref[...])


def _sc_dispatch(x, d0, d1, S, D):
    info = plsc.get_sparse_core_info()
    nw = info.num_cores * info.num_subcores
    ch = S // nw
    mesh = plsc.VectorSubcoreMesh(core_axis_name="c", subcore_axis_name="s")

    @functools.partial(
        pl.kernel, mesh=mesh,
        out_type=jax.ShapeDtypeStruct((NBM, D), jnp.float32),
        scratch_types=[
            pltpu.VMEM((ch,), jnp.int32),
            pltpu.VMEM((ch, D), jnp.float32),
            pltpu.SemaphoreType.DMA,
        ],
    )
    def disp(x_hbm, d0_hbm, d1_hbm, xs_hbm, idx_v, rows_v, sem):
        wid = lax.axis_index("s") * info.num_cores + lax.axis_index("c")
        base = wid * ch
        pltpu.sync_copy(x_hbm.at[pl.ds(base, ch)], rows_v)
        pltpu.sync_copy(d0_hbm.at[pl.ds(base, ch)], idx_v)
        pltpu.async_copy(rows_v, xs_hbm.at[idx_v], sem).wait()
        pltpu.sync_copy(d1_hbm.at[pl.ds(base, ch)], idx_v)
        pltpu.async_copy(rows_v, xs_hbm.at[idx_v], sem).wait()

    return disp(x, d0, d1)


def _sc_combine(ys, d0, d1, S, D):
    info = plsc.get_sparse_core_info()
    nw = info.num_cores * info.num_subcores
    ch = S // nw
    mesh = plsc.VectorSubcoreMesh(core_axis_name="c", subcore_axis_name="s")

    @functools.partial(
        pl.kernel, mesh=mesh,
        out_type=[jax.ShapeDtypeStruct((S, D), jnp.float32),
                  jax.ShapeDtypeStruct((S, D), jnp.float32)],
        scratch_types=[
            pltpu.VMEM((ch,), jnp.int32),
            pltpu.VMEM((ch, D), jnp.float32),
            pltpu.SemaphoreType.DMA,
        ],
    )
    def comb(ys_hbm, d0_hbm, d1_hbm, y0_hbm, y1_hbm, idx_v, rows_v, sem):
        wid = lax.axis_index("s") * info.num_cores + lax.axis_index("c")
        base = wid * ch
        pltpu.sync_copy(d0_hbm.at[pl.ds(base, ch)], idx_v)
        pltpu.async_copy(ys_hbm.at[idx_v], rows_v, sem).wait()
        pltpu.sync_copy(rows_v, y0_hbm.at[pl.ds(base, ch)])
        pltpu.sync_copy(d1_hbm.at[pl.ds(base, ch)], idx_v)
        pltpu.async_copy(ys_hbm.at[idx_v], rows_v, sem).wait()
        pltpu.sync_copy(rows_v, y1_hbm.at[pl.ds(base, ch)])

    return comb(ys, d0, d1)


def _router(xs, Wg, bg):
    S, D = xs.shape
    E = Wg.shape[1]
    return pl.pallas_call(
        _router_body,
        grid=(1,),
        in_specs=[
            pl.BlockSpec((S, D), lambda i: (0, 0)),
            pl.BlockSpec((D, E), lambda i: (0, 0)),
            pl.BlockSpec((1, E), lambda i: (0, 0)),
        ],
        out_specs=[
            pl.BlockSpec((S, 1), lambda i: (0, 0)),
            pl.BlockSpec((S, 1), lambda i: (0, 0)),
            pl.BlockSpec((S, 1), lambda i: (0, 0)),
            pl.BlockSpec((S, 1), lambda i: (0, 0)),
            pl.BlockSpec((1, BLKW), lambda i: (0, 0)),
            pl.BlockSpec((1, BLKW), lambda i: (0, 0)),
        ],
        out_shape=[
            jax.ShapeDtypeStruct((S, 1), jnp.int32),
            jax.ShapeDtypeStruct((S, 1), jnp.int32),
            jax.ShapeDtypeStruct((S, 1), jnp.float32),
            jax.ShapeDtypeStruct((S, 1), jnp.float32),
            jax.ShapeDtypeStruct((1, BLKW), jnp.int32),
            jax.ShapeDtypeStruct((1, BLKW), jnp.int32),
        ],
    )(xs, Wg, bg.reshape(1, E))


def _shared(xs, Ws1, bs1, Ws2, bs2, ib):
    S, D = xs.shape
    I = Ws1.shape[1]
    NI = I // ib
    return pl.pallas_call(
        _shared_body,
        grid=(NI,),
        in_specs=[
            pl.BlockSpec((S, D), lambda i: (0, 0)),
            pl.BlockSpec((D, ib), lambda i: (0, i)),
            pl.BlockSpec((1, ib), lambda i: (0, i)),
            pl.BlockSpec((ib, D), lambda i: (i, 0)),
            pl.BlockSpec((1, D), lambda i: (0, 0)),
        ],
        out_specs=pl.BlockSpec((S, D), lambda i: (0, 0)),
        out_shape=jax.ShapeDtypeStruct((S, D), jnp.float32),
    )(xs, Ws1, bs1.reshape(1, I), Ws2, bs2.reshape(1, D))


def _expert_blocks(xsorted, We1, be1, We2, be2, blke, blka):
    E, D, I = We1.shape
    grid_spec = pltpu.PrefetchScalarGridSpec(
        num_scalar_prefetch=2,
        grid=(NB,),
        in_specs=[
            pl.BlockSpec((M, D), lambda b, be, ba: (b, 0)),
            pl.BlockSpec((1, D, I), lambda b, be, ba: (be[b], 0, 0)),
            pl.BlockSpec((1, 1, I), lambda b, be, ba: (be[b], 0, 0)),
            pl.BlockSpec((1, I, D), lambda b, be, ba: (be[b], 0, 0)),
            pl.BlockSpec((1, 1, D), lambda b, be, ba: (be[b], 0, 0)),
        ],
        out_specs=pl.BlockSpec((M, D), lambda b, be, ba: (b, 0)),
    )
    return pl.pallas_call(
        _expert_body,
        grid_spec=grid_spec,
        out_shape=jax.ShapeDtypeStruct((NBM, D), jnp.float32),
    )(blke, blka, xsorted, We1, be1.reshape(E, 1, I), We2,
      be2.reshape(E, 1, D))


def _final(partial, y0, y1, gw0, gw1, bt):
    S, D = partial.shape
    return pl.pallas_call(
        _final_body,
        grid=(S // bt,),
        in_specs=[
            pl.BlockSpec((bt, D), lambda t: (t, 0)),
            pl.BlockSpec((bt, D), lambda t: (t, 0)),
            pl.BlockSpec((bt, D), lambda t: (t, 0)),
            pl.BlockSpec((bt, 1), lambda t: (t, 0)),
            pl.BlockSpec((bt, 1), lambda t: (t, 0)),
        ],
        out_specs=pl.BlockSpec((bt, D), lambda t: (t, 0)),
        out_shape=jax.ShapeDtypeStruct((S, D), jnp.float32),
    )(partial, y0, y1, gw0, gw1)


def kernel(x, Wg, bg, Ws1, bs1, Ws2, bs2, We1, be1, We2, be2):
    B, S, D = x.shape
    I = Ws1.shape[1]
    xs = x.reshape(S, D)
    d0, d1, gw0, gw1, blke, blka = _router(xs, Wg, bg)
    xsorted = _sc_dispatch(xs, d0.reshape(S), d1.reshape(S), S, D)
    partial = _shared(xs, Ws1, bs1, Ws2, bs2, min(I, 512))
    ysorted = _expert_blocks(xsorted, We1, be1, We2, be2,
                             blke.reshape(BLKW), blka.reshape(BLKW))
    y0, y1 = _sc_combine(ysorted, d0.reshape(S), d1.reshape(S), S, D)
    out = _final(partial, y0, y1, gw0, gw1, min(S, 1024))
    return out.reshape(B, S, D)


# speedup vs baseline: 1.4037x; 1.0150x over previous
import functools

import jax
import jax.numpy as jnp
from jax import lax
from jax.experimental import pallas as pl
from jax.experimental.pallas import tpu as pltpu
from jax.experimental.pallas import tpu_sc as plsc

M = 512
NB = 15
NBM = NB * M
BLKW = 16


def _router_body(x_ref, wg_ref, bg_ref,
                 d0_ref, d1_ref, gw0_ref, gw1_ref, blke_ref, blka_ref):
    x = x_ref[...]
    S = x.shape[0]
    E = wg_ref.shape[1]
    logits = jnp.dot(x, wg_ref[...], preferred_element_type=jnp.float32)
    logits = logits + bg_ref[...]
    m = jnp.max(logits, axis=-1, keepdims=True)
    p = jnp.exp(logits - m)
    p = p / jnp.sum(p, axis=-1, keepdims=True)
    top1 = jnp.max(p, axis=-1, keepdims=True)
    is1 = p == top1
    p2 = jnp.where(is1, -jnp.inf, p)
    top2 = jnp.max(p2, axis=-1, keepdims=True)
    is2 = p2 == top2
    denom = top1 + top2 + 1e-9
    gw0_ref[...] = top1 / denom
    gw1_ref[...] = top2 / denom

    m0 = is1.astype(jnp.float32)
    m1 = is2.astype(jnp.float32)
    r = lax.broadcasted_iota(jnp.int32, (128, 128), 0)
    c = lax.broadcasted_iota(jnp.int32, (128, 128), 1)
    lstrict = (r < c).astype(jnp.float32)

    def ranks(mm):
        run = jnp.zeros((1, E), jnp.float32)
        parts = []
        for i in range(S // 128):
            ch = mm[128 * i:128 * (i + 1), :]
            pre = jnp.dot(jnp.transpose(lstrict),
                          ch, preferred_element_type=jnp.float32)
            parts.append(pre + run)
            run = run + jnp.sum(ch, axis=0, keepdims=True)
        return jnp.concatenate(parts, axis=0), run

    rank0, cnt0 = ranks(m0)
    rank1, cnt1 = ranks(m1)
    rank1 = rank1 + cnt0
    counts = cnt0 + cnt1
    nblk = jnp.ceil(counts / M)
    l8r = lax.broadcasted_iota(jnp.int32, (E, E), 0)
    l8c = lax.broadcasted_iota(jnp.int32, (E, E), 1)
    l8 = (l8r < l8c).astype(jnp.float32)
    excl_blk = jnp.dot(nblk, l8, preferred_element_type=jnp.float32)
    offs = M * excl_blk
    d0 = offs + rank0
    d1 = offs + rank1
    d0_ref[...] = jnp.sum(jnp.where(is1, d0, 0.0), axis=-1,
                          keepdims=True).astype(jnp.int32)
    d1_ref[...] = jnp.sum(jnp.where(is2, d1, 0.0), axis=-1,
                          keepdims=True).astype(jnp.int32)

    end_blk = excl_blk + nblk
    total_blk = end_blk[:, E - 1:E]
    biota = lax.broadcasted_iota(
        jnp.int32, (1, blke_ref.shape[1]), 1).astype(jnp.float32)
    blke = jnp.zeros_like(biota)
    for e in range(E):
        end_e = end_blk[:, e:e + 1]
        blke = blke + (biota >= end_e).astype(jnp.float32)
    blke_ref[...] = jnp.minimum(blke, float(E - 1)).astype(jnp.int32)
    blka_ref[...] = (biota < total_blk).astype(jnp.int32)


def _shared_body(x_ref, w1_ref, b1_ref, w2_ref, b2_ref, out_ref):
    i2 = pl.program_id(0)

    @pl.when(i2 == 0)
    def _():
        out_ref[...] = x_ref[...] + b2_ref[...]

    h = jax.nn.gelu(
        jnp.dot(x_ref[...].astype(jnp.bfloat16),
                w1_ref[...].astype(jnp.bfloat16),
                preferred_element_type=jnp.float32)
        + b1_ref[...])
    out_ref[...] += jnp.dot(h.astype(jnp.bfloat16),
                            w2_ref[...].astype(jnp.bfloat16),
                            preferred_element_type=jnp.float32)


def _expert_body(blke_ref, blka_ref, x_ref, w1_ref, b1_ref, w2_ref, b2_ref,
                 out_ref):
    b = pl.program_id(0)

    @pl.when(blka_ref[b] != 0)
    def _():
        h = jax.nn.gelu(
            jnp.dot(x_ref[...].astype(jnp.bfloat16),
                    w1_ref[0].astype(jnp.bfloat16),
                    preferred_element_type=jnp.float32)
            + b1_ref[0])
        y = jnp.dot(h.astype(jnp.bfloat16), w2_ref[0].astype(jnp.bfloat16),
                    preferred_element_type=jnp.float32)
        out_ref[...] = y + b2_ref[0]


def _final_body(p_ref, y0_ref, y1_ref, g0_ref, g1_ref, out_ref):
    out_ref[...] = (p_ref[...] + g0_ref[...] * y0_ref[...]
                    + g1_ref[...] * y1_ref[...])


def _sc_dispatch(x, d0, d1, S, D):
    info = plsc.get_sparse_core_info()
    nw = info.num_cores * info.num_subcores
    ch = S // nw
    mesh = plsc.VectorSubcoreMesh(core_axis_name="c", subcore_axis_name="s")

    @functools.partial(
        pl.kernel, mesh=mesh,
        out_type=jax.ShapeDtypeStruct((NBM, D), jnp.float32),
        scratch_types=[
            pltpu.VMEM((ch,), jnp.int32),
            pltpu.VMEM((ch, D), jnp.float32),
            pltpu.SemaphoreType.DMA,
        ],
    )
    def disp(x_hbm, d0_hbm, d1_hbm, xs_hbm, idx_v, rows_v, sem):
        wid = lax.axis_index("s") * info.num_cores + lax.axis_index("c")
        base = wid * ch
        pltpu.sync_copy(x_hbm.at[pl.ds(base, ch)], rows_v)
        pltpu.sync_copy(d0_hbm.at[pl.ds(base, ch)], idx_v)
        pltpu.async_copy(rows_v, xs_hbm.at[idx_v], sem).wait()
        pltpu.sync_copy(d1_hbm.at[pl.ds(base, ch)], idx_v)
        pltpu.async_copy(rows_v, xs_hbm.at[idx_v], sem).wait()

    return disp(x, d0, d1)


def _sc_combine(ys, d0, d1, S, D):
    info = plsc.get_sparse_core_info()
    nw = info.num_cores * info.num_subcores
    ch = S // nw
    mesh = plsc.VectorSubcoreMesh(core_axis_name="c", subcore_axis_name="s")

    @functools.partial(
        pl.kernel, mesh=mesh,
        out_type=[jax.ShapeDtypeStruct((S, D), jnp.float32),
                  jax.ShapeDtypeStruct((S, D), jnp.float32)],
        scratch_types=[
            pltpu.VMEM((ch,), jnp.int32),
            pltpu.VMEM((ch, D), jnp.float32),
            pltpu.SemaphoreType.DMA,
        ],
    )
    def comb(ys_hbm, d0_hbm, d1_hbm, y0_hbm, y1_hbm, idx_v, rows_v, sem):
        wid = lax.axis_index("s") * info.num_cores + lax.axis_index("c")
        base = wid * ch
        pltpu.sync_copy(d0_hbm.at[pl.ds(base, ch)], idx_v)
        pltpu.async_copy(ys_hbm.at[idx_v], rows_v, sem).wait()
        pltpu.sync_copy(rows_v, y0_hbm.at[pl.ds(base, ch)])
        pltpu.sync_copy(d1_hbm.at[pl.ds(base, ch)], idx_v)
        pltpu.async_copy(ys_hbm.at[idx_v], rows_v, sem).wait()
        pltpu.sync_copy(rows_v, y1_hbm.at[pl.ds(base, ch)])

    return comb(ys, d0, d1)


def _router(xs, Wg, bg):
    S, D = xs.shape
    E = Wg.shape[1]
    return pl.pallas_call(
        _router_body,
        grid=(1,),
        in_specs=[
            pl.BlockSpec((S, D), lambda i: (0, 0)),
            pl.BlockSpec((D, E), lambda i: (0, 0)),
            pl.BlockSpec((1, E), lambda i: (0, 0)),
        ],
        out_specs=[
            pl.BlockSpec((S, 1), lambda i: (0, 0)),
            pl.BlockSpec((S, 1), lambda i: (0, 0)),
            pl.BlockSpec((S, 1), lambda i: (0, 0)),
            pl.BlockSpec((S, 1), lambda i: (0, 0)),
            pl.BlockSpec((1, BLKW), lambda i: (0, 0)),
            pl.BlockSpec((1, BLKW), lambda i: (0, 0)),
        ],
        out_shape=[
            jax.ShapeDtypeStruct((S, 1), jnp.int32),
            jax.ShapeDtypeStruct((S, 1), jnp.int32),
            jax.ShapeDtypeStruct((S, 1), jnp.float32),
            jax.ShapeDtypeStruct((S, 1), jnp.float32),
            jax.ShapeDtypeStruct((1, BLKW), jnp.int32),
            jax.ShapeDtypeStruct((1, BLKW), jnp.int32),
        ],
    )(xs, Wg, bg.reshape(1, E))


def _shared(xs, Ws1, bs1, Ws2, bs2, ib):
    S, D = xs.shape
    I = Ws1.shape[1]
    NI = I // ib
    return pl.pallas_call(
        _shared_body,
        grid=(NI,),
        in_specs=[
            pl.BlockSpec((S, D), lambda i: (0, 0)),
            pl.BlockSpec((D, ib), lambda i: (0, i)),
            pl.BlockSpec((1, ib), lambda i: (0, i)),
            pl.BlockSpec((ib, D), lambda i: (i, 0)),
            pl.BlockSpec((1, D), lambda i: (0, 0)),
        ],
        out_specs=pl.BlockSpec((S, D), lambda i: (0, 0)),
        out_shape=jax.ShapeDtypeStruct((S, D), jnp.float32),
    )(xs, Ws1, bs1.reshape(1, I), Ws2, bs2.reshape(1, D))


def _expert_blocks(xsorted, We1, be1, We2, be2, blke, blka):
    E, D, I = We1.shape
    grid_spec = pltpu.PrefetchScalarGridSpec(
        num_scalar_prefetch=2,
        grid=(NB,),
        in_specs=[
            pl.BlockSpec((M, D), lambda b, be, ba: (ba[b] * b, 0)),
            pl.BlockSpec((1, D, I), lambda b, be, ba: (be[b], 0, 0)),
            pl.BlockSpec((1, 1, I), lambda b, be, ba: (be[b], 0, 0)),
            pl.BlockSpec((1, I, D), lambda b, be, ba: (be[b], 0, 0)),
            pl.BlockSpec((1, 1, D), lambda b, be, ba: (be[b], 0, 0)),
        ],
        out_specs=pl.BlockSpec(
            (M, D),
            lambda b, be, ba: (ba[b] * b + (1 - ba[b]) * (NB - 1), 0)),
    )
    return pl.pallas_call(
        _expert_body,
        grid_spec=grid_spec,
        out_shape=jax.ShapeDtypeStruct((NBM, D), jnp.float32),
    )(blke, blka, xsorted, We1, be1.reshape(E, 1, I), We2,
      be2.reshape(E, 1, D))


def _final(partial, y0, y1, gw0, gw1, bt):
    S, D = partial.shape
    return pl.pallas_call(
        _final_body,
        grid=(S // bt,),
        in_specs=[
            pl.BlockSpec((bt, D), lambda t: (t, 0)),
            pl.BlockSpec((bt, D), lambda t: (t, 0)),
            pl.BlockSpec((bt, D), lambda t: (t, 0)),
            pl.BlockSpec((bt, 1), lambda t: (t, 0)),
            pl.BlockSpec((bt, 1), lambda t: (t, 0)),
        ],
        out_specs=pl.BlockSpec((bt, D), lambda t: (t, 0)),
        out_shape=jax.ShapeDtypeStruct((S, D), jnp.float32),
    )(partial, y0, y1, gw0, gw1)


def kernel(x, Wg, bg, Ws1, bs1, Ws2, bs2, We1, be1, We2, be2):
    B, S, D = x.shape
    I = Ws1.shape[1]
    xs = x.reshape(S, D)
    d0, d1, gw0, gw1, blke, blka = _router(xs, Wg, bg)
    xsorted = _sc_dispatch(xs, d0.reshape(S), d1.reshape(S), S, D)
    partial = _shared(xs, Ws1, bs1, Ws2, bs2, min(I, 512))
    ysorted = _expert_blocks(xsorted, We1, be1, We2, be2,
                             blke.reshape(BLKW), blka.reshape(BLKW))
    y0, y1 = _sc_combine(ysorted, d0.reshape(S), d1.reshape(S), S, D)
    out = _final(partial, y0, y1, gw0, gw1, min(S, 1024))
    return out.reshape(B, S, D)
